# baseline (speedup 1.0000x reference)
import functools

import jax
import jax.numpy as jnp
from jax import lax
from jax.experimental import pallas as pl
from jax.experimental.pallas import tpu as pltpu
from jax.experimental.pallas import tpu_sc as plsc

B, T, N, C, K = 8, 32, 256, 768, 10
R = B * K
ROW = N * C
NCH = 4
CHUNK = ROW // NCH
NW = 32
TPW = R * NCH // NW
APW = R // 16


def _body(idx_hbm, patch_hbm, audio_hbm, out_patch, out_audio,
          idx_v, sem, asem):
    c = lax.axis_index("c")
    s = lax.axis_index("s")
    w = s * 2 + c

    pltpu.sync_copy(idx_hbm, idx_v)
    iota = lax.iota(jnp.int32, 16)

    def extract(r):
        base = pl.multiple_of((r // 16) * 16, 16)
        vec = idx_v[pl.ds(base, 16)]
        return jnp.max(jnp.where(iota == r % 16, vec, 0))

    handles = []
    for j in range(TPW):
        g = w * TPW + j
        r = g // NCH
        ch = g % NCH
        tval = extract(r)
        src_row = (r // K) * T + tval
        handles.append(pltpu.async_copy(
            patch_hbm.at[src_row, pl.ds(ch * CHUNK, CHUNK)],
            out_patch.at[r, pl.ds(ch * CHUNK, CHUNK)], sem))

    @pl.when(w < 16)
    def _audio():
        ahandles = []
        for j in range(APW):
            r = w * APW + j
            tval = extract(r)
            src_row = (r // K) * T + tval
            ahandles.append(pltpu.async_copy(
                audio_hbm.at[src_row], out_audio.at[r], asem))
        for h in ahandles:
            h.wait()

    for h in handles:
        h.wait()


@jax.jit
def _gather_call(idx, patch2d, audio2d):
    mesh = plsc.VectorSubcoreMesh(core_axis_name="c", subcore_axis_name="s")
    run = functools.partial(
        pl.kernel,
        mesh=mesh,
        compiler_params=pltpu.CompilerParams(needs_layout_passes=False),
        out_type=(
            jax.ShapeDtypeStruct((R, ROW), jnp.float32),
            jax.ShapeDtypeStruct((R, C), jnp.float32),
        ),
        scratch_types=[
            pltpu.VMEM((R,), jnp.int32),
            pltpu.SemaphoreType.DMA,
            pltpu.SemaphoreType.DMA,
        ],
    )(_body)
    return run(idx, patch2d, audio2d)


def kernel(top_k_index_sort, patch_feat, audio_feat):
    idx = top_k_index_sort.reshape(R).astype(jnp.int32)
    patch2d = patch_feat.reshape(B * T, ROW)
    audio2d = audio_feat.reshape(B * T, C)
    out_p, out_a = _gather_call(idx, patch2d, audio2d)
    return out_p.reshape(B, K, N, C), out_a.reshape(B, K, C)


# speedup vs baseline: 7.1077x; 7.1077x over previous
import functools

import jax
import jax.numpy as jnp
from jax import lax
from jax.experimental import pallas as pl
from jax.experimental.pallas import tpu as pltpu
from jax.experimental.pallas import tpu_sc as plsc

B, T, N, C, K = 8, 32, 256, 768, 10
R = B * K
ROW = N * C
NCH = 4
CHUNK = ROW // NCH
NW = 32
TPW = R * NCH // NW
APW = R // 16


def _body(idx_hbm, patch_hbm, audio_hbm, out_patch, out_audio,
          idx_v, bufs, abuf, rs0, rs1, ws0, ws1, asem):
    c = lax.axis_index("c")
    s = lax.axis_index("s")
    w = s * 2 + c

    pltpu.sync_copy(idx_hbm, idx_v)
    iota = lax.iota(jnp.int32, 16)

    def extract(r):
        base = pl.multiple_of((r // 16) * 16, 16)
        vec = idx_v[pl.ds(base, 16)]
        return jnp.max(jnp.where(iota == r % 16, vec, 0))

    @pl.when(w < 16)
    def _audio_in():
        for j in range(APW):
            r = w * APW + j
            src_row = (r // K) * T + extract(r)
            pltpu.async_copy(audio_hbm.at[pl.ds(src_row, 1)],
                             abuf.at[j], asem)

    rsem = [rs0, rs1]
    wsem = [ws0, ws1]
    rh = [None, None]
    wh = [None, None]

    def read(j):
        g = w * TPW + j
        r = g // NCH
        ch = g % NCH
        src_row = ((r // K) * T + extract(r)) * NCH + ch
        return pltpu.async_copy(patch_hbm.at[pl.ds(src_row, 1)],
                                bufs.at[j % 2], rsem[j % 2])

    def write(j):
        g = w * TPW + j
        return pltpu.async_copy(bufs.at[j % 2],
                                out_patch.at[pl.ds(g, 1)], wsem[j % 2])

    rh[0] = read(0)
    for j in range(1, TPW):
        if wh[j % 2] is not None:
            wh[j % 2].wait()
        rh[j % 2] = read(j)
        rh[(j - 1) % 2].wait()
        wh[(j - 1) % 2] = write(j - 1)
    rh[(TPW - 1) % 2].wait()
    wh[(TPW - 1) % 2] = write(TPW - 1)
    wh[0].wait()
    wh[1].wait()

    @pl.when(w < 16)
    def _audio_out():
        for j in range(APW):
            pltpu.make_async_copy(audio_hbm.at[pl.ds(0, 1)],
                                  abuf.at[j], asem).wait()
            pltpu.sync_copy(abuf.at[j],
                            out_audio.at[pl.ds(w * APW + j, 1)])


@jax.jit
def _gather_call(idx, patch2d, audio2d):
    mesh = plsc.VectorSubcoreMesh(core_axis_name="c", subcore_axis_name="s")
    run = functools.partial(
        pl.kernel,
        mesh=mesh,
        compiler_params=pltpu.CompilerParams(needs_layout_passes=False),
        out_type=(
            jax.ShapeDtypeStruct((R * NCH, CHUNK), jnp.float32),
            jax.ShapeDtypeStruct((R, C), jnp.float32),
        ),
        scratch_types=[
            pltpu.VMEM((R,), jnp.int32),
            pltpu.VMEM((2, 1, CHUNK), jnp.float32),
            pltpu.VMEM((APW, 1, C), jnp.float32),
            pltpu.SemaphoreType.DMA,
            pltpu.SemaphoreType.DMA,
            pltpu.SemaphoreType.DMA,
            pltpu.SemaphoreType.DMA,
            pltpu.SemaphoreType.DMA,
        ],
    )(_body)
    return run(idx, patch2d, audio2d)


def kernel(top_k_index_sort, patch_feat, audio_feat):
    idx = top_k_index_sort.reshape(R).astype(jnp.int32)
    patch2d = patch_feat.reshape(B * T * NCH, CHUNK)
    audio2d = audio_feat.reshape(B * T, C)
    out_p, out_a = _gather_call(idx, patch2d, audio2d)
    return out_p.reshape(B, K, N, C), out_a.reshape(B, K, C)


# speedup vs baseline: 7.1645x; 1.0080x over previous
import functools

import jax
import jax.numpy as jnp
from jax import lax
from jax.experimental import pallas as pl
from jax.experimental.pallas import tpu as pltpu
from jax.experimental.pallas import tpu_sc as plsc

B, T, N, C, K = 8, 32, 256, 768, 10
R = B * K
ROW = N * C
NCH = 4
CHUNK = ROW // NCH
NW = 32
TPW = R * NCH // NW
APW = R // 16


def _body(idx_hbm, patch_hbm, audio_hbm, out_patch, out_audio,
          idx_v, bufs, abuf, rs0, rs1, ws0, ws1, asem):
    c = lax.axis_index("c")
    s = lax.axis_index("s")
    w = s * 2 + c

    pltpu.sync_copy(idx_hbm, idx_v)
    iota = lax.iota(jnp.int32, 16)

    def extract(r):
        base = pl.multiple_of((r // 16) * 16, 16)
        vec = idx_v[pl.ds(base, 16)]
        return jnp.max(jnp.where(iota == r % 16, vec, 0))

    @pl.when(w < 16)
    def _audio_in():
        for j in range(APW):
            r = w * APW + j
            src_row = (r // K) * T + extract(r)
            pltpu.async_copy(audio_hbm.at[pl.ds(src_row, 1)],
                             abuf.at[j], asem)

    rsem = [rs0, rs1]
    wsem = [ws0, ws1]
    rh = [None, None]
    wh = [None, None]

    def read(j):
        g = w * TPW + j
        r = g // NCH
        ch = g % NCH
        src_row = ((r // K) * T + extract(r)) * NCH + ch
        return pltpu.async_copy(patch_hbm.at[pl.ds(src_row, 1)],
                                bufs.at[s, j % 2], rsem[j % 2])

    def write(j):
        g = w * TPW + j
        return pltpu.async_copy(bufs.at[s, j % 2],
                                out_patch.at[pl.ds(g, 1)], wsem[j % 2])

    rh[0] = read(0)
    for j in range(1, TPW):
        if wh[j % 2] is not None:
            wh[j % 2].wait()
        rh[j % 2] = read(j)
        rh[(j - 1) % 2].wait()
        wh[(j - 1) % 2] = write(j - 1)
    rh[(TPW - 1) % 2].wait()
    wh[(TPW - 1) % 2] = write(TPW - 1)
    wh[0].wait()
    wh[1].wait()

    @pl.when(w < 16)
    def _audio_out():
        for j in range(APW):
            pltpu.make_async_copy(audio_hbm.at[pl.ds(0, 1)],
                                  abuf.at[j], asem).wait()
            pltpu.sync_copy(abuf.at[j],
                            out_audio.at[pl.ds(w * APW + j, 1)])


@jax.jit
def _gather_call(idx, patch2d, audio2d):
    mesh = plsc.VectorSubcoreMesh(core_axis_name="c", subcore_axis_name="s")
    run = functools.partial(
        pl.kernel,
        mesh=mesh,
        compiler_params=pltpu.CompilerParams(needs_layout_passes=False),
        out_type=(
            jax.ShapeDtypeStruct((R * NCH, CHUNK), jnp.float32),
            jax.ShapeDtypeStruct((R, C), jnp.float32),
        ),
        scratch_types=[
            pltpu.VMEM((R,), jnp.int32),
            pltpu.VMEM_SHARED((16, 2, 1, CHUNK), jnp.float32),
            pltpu.VMEM((APW, 1, C), jnp.float32),
            pltpu.SemaphoreType.DMA,
            pltpu.SemaphoreType.DMA,
            pltpu.SemaphoreType.DMA,
            pltpu.SemaphoreType.DMA,
            pltpu.SemaphoreType.DMA,
        ],
    )(_body)
    return run(idx, patch2d, audio2d)


def kernel(top_k_index_sort, patch_feat, audio_feat):
    idx = top_k_index_sort.reshape(R).astype(jnp.int32)
    patch2d = patch_feat.reshape(B * T * NCH, CHUNK)
    audio2d = audio_feat.reshape(B * T, C)
    out_p, out_a = _gather_call(idx, patch2d, audio2d)
    return out_p.reshape(B, K, N, C), out_a.reshape(B, K, C)
